# baseline (speedup 1.0000x reference)
import functools

import jax
import jax.numpy as jnp
from jax import lax
from jax.experimental import pallas as pl
from jax.experimental.pallas import tpu as pltpu
from jax.experimental.pallas import tpu_sc as plsc

N = 10000
E = 320000
D = 128
G = 128
NP = 10240
NC = 2
NS = 16
NW = NC * NS
EPT = 10240
K = 128
NCH = EPT // K
BR = 512
NBLK = NP // BR

_mesh = functools.partial(
    plsc.VectorSubcoreMesh, core_axis_name="c", subcore_axis_name="s")


def _deg_body(dst_hbm, deg_out, dstbuf, acc):
  c = lax.axis_index("c")
  s = lax.axis_index("s")
  w = s * NC + c

  def zero(i, _):
    acc[pl.ds(i * 16, 16)] = jnp.zeros((16,), jnp.float32)
    return 0
  lax.fori_loop(0, NP // 16, zero, 0)

  pltpu.sync_copy(dst_hbm.at[w], dstbuf)
  ones = jnp.ones((16,), jnp.float32)

  def body(j, _):
    idx = dstbuf[pl.ds(j * 16, 16)]
    plsc.addupdate_scatter(acc, [idx], ones)
    return 0
  lax.fori_loop(0, EPT // 16, body, 0)
  pltpu.sync_copy(acc, deg_out.at[w])


def _deg_call(dst_t):
  kern = pl.kernel(
      _deg_body,
      out_type=jax.ShapeDtypeStruct((NW, NP), jnp.float32),
      mesh=_mesh(),
      scratch_types=[
          pltpu.VMEM((EPT,), jnp.int32),
          pltpu.VMEM((NP,), jnp.float32),
      ],
      compiler_params=pltpu.CompilerParams(needs_layout_passes=False),
  )
  return kern(dst_t)


def _h2_body(x_ref, w_ref, deg_ref, batch_ref, h2_ref, starts_ref):
  ones = jnp.ones((NW, 1), jnp.float32)
  degs = lax.dot_general(deg_ref[...], ones,
                         (((0,), (0,)), ((), ()))) + 1.0
  dinv = lax.rsqrt(degs)
  h = jnp.dot(x_ref[...], w_ref[...], preferred_element_type=jnp.float32)
  h2_ref[...] = h * dinv

  @pl.when(pl.program_id(0) == 0)
  def _():
    b = batch_ref[...]
    gid = lax.broadcasted_iota(jnp.int32, (1, G), 1)
    oh = (b == gid).astype(jnp.float32)
    hist = jnp.sum(oh, axis=0)
    r = lax.broadcasted_iota(jnp.int32, (G, G), 0)
    cc = lax.broadcasted_iota(jnp.int32, (G, G), 1)
    ut = (r < cc).astype(jnp.float32)
    starts = lax.dot_general(hist, ut, (((0,), (0,)), ((), ())))
    starts_i = starts.astype(jnp.int32)
    first = jnp.minimum(starts_i, N - 1)
    pad = jnp.full((6, G), N, jnp.int32)
    starts_ref[...] = jnp.concatenate(
        [starts_i[None], first[None], pad], axis=0)


def _h2_call(x_pad, W_conv, deg_part, batch_col):
  return pl.pallas_call(
      _h2_body,
      grid=(NBLK,),
      in_specs=[
          pl.BlockSpec((BR, D), lambda i: (i, 0)),
          pl.BlockSpec((D, D), lambda i: (0, 0)),
          pl.BlockSpec((NW, BR), lambda i: (0, i)),
          pl.BlockSpec((NP, 1), lambda i: (0, 0)),
      ],
      out_specs=[
          pl.BlockSpec((BR, D), lambda i: (i, 0)),
          pl.BlockSpec((8, G), lambda i: (0, 0)),
      ],
      out_shape=[
          jax.ShapeDtypeStruct((NP, D), jnp.float32),
          jax.ShapeDtypeStruct((8, G), jnp.int32),
      ],
  )(x_pad, W_conv, deg_part, batch_col)


def _agg_body(h2_hbm, src_hbm, dst_hbm, zrows_hbm, p0, p1,
              src_v, dst_v, rows, acc, gsem):
  c = lax.axis_index("c")
  s = lax.axis_index("s")
  w = s * NC + c
  rpt = NP // NS
  r0 = s * rpt
  zk = 128

  for k in range(rpt // zk):
    pltpu.sync_copy(zrows_hbm, acc.at[pl.ds(r0 + k * zk, zk)])
  plsc.subcore_barrier()

  def chunk(j, _):
    base = j * K
    pltpu.sync_copy(src_hbm.at[w, pl.ds(base, K)], src_v)
    pltpu.sync_copy(dst_hbm.at[w, pl.ds(base, K)], dst_v)
    pltpu.async_copy(h2_hbm.at[src_v], rows, gsem).wait()
    pltpu.sync_copy(rows, acc.at[dst_v], add=True)
    return 0
  lax.fori_loop(0, NCH, chunk, 0)
  plsc.subcore_barrier()

  sl = pl.ds(r0, rpt)

  @pl.when(c == 0)
  def _():
    pltpu.sync_copy(acc.at[sl], p0.at[sl])

  @pl.when(c == 1)
  def _():
    pltpu.sync_copy(acc.at[sl], p1.at[sl])


def _agg_call(h2, src_t, dst_t, zrows):
  kern = pl.kernel(
      _agg_body,
      out_type=(
          jax.ShapeDtypeStruct((NP, D), jnp.float32),
          jax.ShapeDtypeStruct((NP, D), jnp.float32),
      ),
      mesh=_mesh(),
      scratch_types=[
          pltpu.VMEM((K,), jnp.int32),
          pltpu.VMEM((K,), jnp.int32),
          pltpu.VMEM((K, D), jnp.float32),
          pltpu.VMEM_SHARED((NP, D), jnp.float32),
          pltpu.SemaphoreType.DMA,
      ],
      compiler_params=pltpu.CompilerParams(needs_layout_passes=False),
  )
  return kern(h2, src_t, dst_t, zrows)


def _x1_body(p0_ref, p1_ref, h2_ref, deg_ref, b_ref, x1_ref):
  ones = jnp.ones((NW, 1), jnp.float32)
  degs = lax.dot_general(deg_ref[...], ones,
                         (((0,), (0,)), ((), ()))) + 1.0
  dinv = lax.rsqrt(degs)
  agg = p0_ref[...] + p1_ref[...] + h2_ref[...]
  x1_ref[...] = jnp.maximum(dinv * agg + b_ref[...], 0.0)


def _x1_call(p0, p1, h2, deg_part, b_conv_row):
  blk = pl.BlockSpec((BR, D), lambda i: (i, 0))
  return pl.pallas_call(
      _x1_body,
      grid=(NBLK,),
      in_specs=[
          blk,
          blk,
          blk,
          pl.BlockSpec((NW, BR), lambda i: (0, i)),
          pl.BlockSpec((1, D), lambda i: (0, 0)),
      ],
      out_specs=pl.BlockSpec((BR, D), lambda i: (i, 0)),
      out_shape=jax.ShapeDtypeStruct((NP, D), jnp.float32),
  )(p0, p1, h2, deg_part, b_conv_row)


def _pool_body(x1_hbm, x_hbm, starts_hbm, first_hbm, pooled, news,
               starts_v, first_v, buf, rowbuf, newsbuf):
  c = lax.axis_index("c")
  s = lax.axis_index("s")
  w = s * NC + c
  pltpu.sync_copy(starts_hbm, starts_v)
  pltpu.sync_copy(first_hbm, first_v)
  lane = lax.iota(jnp.int32, 16)

  def extract(ref, idx):
    vec = ref[pl.ds((idx // 16) * 16, 16)]
    return jnp.max(jnp.where(lane == (idx % 16), vec, -1))

  for jj in range(G // NW):
    g = w * (G // NW) + jj
    sval = extract(starts_v, g)
    eval_ = extract(starts_v, g + 1)
    fi = extract(first_v, g)

    pltpu.sync_copy(x_hbm.at[fi], newsbuf)
    pltpu.sync_copy(newsbuf, news.at[g])

    rowa = (sval // K) * K
    init_m = tuple(jnp.full((16,), -jnp.inf, jnp.float32) for _ in range(8))

    def cond(carry):
      return rowa + carry[0] * K < eval_

    def body(carry):
      k, m = carry
      row0 = rowa + k * K
      pltpu.sync_copy(x1_hbm.at[pl.ds(row0, K)], buf)

      def inner(r, m):
        gr = row0 + r
        valid = jnp.logical_and(gr >= sval, gr < eval_)
        return tuple(
            jnp.where(valid,
                      jnp.maximum(m[cc], buf[r, pl.ds(cc * 16, 16)]),
                      m[cc])
            for cc in range(8))
      m = lax.fori_loop(0, K, inner, m)
      return (k + 1, m)

    _, m = lax.while_loop(cond, body, (jnp.int32(0), init_m))
    for cc in range(8):
      rowbuf[pl.ds(cc * 16, 16)] = m[cc]
    pltpu.sync_copy(rowbuf, pooled.at[g])


def _pool_call(x1, x_pad, starts_ext, first_idx):
  kern = pl.kernel(
      _pool_body,
      out_type=(
          jax.ShapeDtypeStruct((G, D), jnp.float32),
          jax.ShapeDtypeStruct((G, D), jnp.float32),
      ),
      mesh=_mesh(),
      scratch_types=[
          pltpu.VMEM((144,), jnp.int32),
          pltpu.VMEM((G,), jnp.int32),
          pltpu.VMEM((K, D), jnp.float32),
          pltpu.VMEM((D,), jnp.float32),
          pltpu.VMEM((D,), jnp.float32),
      ],
      compiler_params=pltpu.CompilerParams(needs_layout_passes=False),
  )
  return kern(x1, x_pad, starts_ext, first_idx)


def _head_body(pooled_ref, newsr_ref, w0_ref, b0_ref, w1a_ref, w1b_ref,
               b1_ref, w2_ref, b2_ref, out_ref):
  nr = jnp.dot(newsr_ref[...], w0_ref[...],
               preferred_element_type=jnp.float32) + b0_ref[...]
  nr = jnp.maximum(nr, 0.0)
  z = (jnp.dot(pooled_ref[...], w1a_ref[...],
               preferred_element_type=jnp.float32)
       + jnp.dot(nr, w1b_ref[...], preferred_element_type=jnp.float32)
       + b1_ref[...])
  z = jnp.maximum(z, 0.0)
  logits = jnp.dot(z, w2_ref[...],
                   preferred_element_type=jnp.float32) + b2_ref[...]
  colmask = lax.broadcasted_iota(jnp.int32, (G, D), 1) < 2
  lm = jnp.where(colmask, logits, -1e30)
  mx = jnp.max(lm, axis=1, keepdims=True)
  sh = lm - mx
  se = jnp.sum(jnp.exp(sh), axis=1, keepdims=True)
  out_ref[...] = sh - jnp.log(se)


def _head_call(pooled, news_rows, W0, b0r, W1a, W1b, b1r, W2p, b2r):
  return pl.pallas_call(
      _head_body,
      out_shape=jax.ShapeDtypeStruct((G, D), jnp.float32),
  )(pooled, news_rows, W0, b0r, W1a, W1b, b1r, W2p, b2r)


@jax.jit
def kernel(x, edge_index, batch, num_graphs,
           W_conv, b_conv, W0, b0, W1, b1, W2, b2):
  del num_graphs
  src = edge_index[0].astype(jnp.int32).reshape(NW, E // NW)
  dst = edge_index[1].astype(jnp.int32).reshape(NW, E // NW)
  padw = EPT - E // NW
  src_t = jnp.concatenate(
      [src, jnp.zeros((NW, padw), jnp.int32)], axis=1)
  dst_pad = jnp.broadcast_to(
      N + jnp.arange(padw, dtype=jnp.int32) % (NP - N), (NW, padw))
  dst_t = jnp.concatenate([dst, dst_pad], axis=1)
  x_pad = jnp.pad(x, ((0, NP - N), (0, 0)))
  batch_col = jnp.pad(batch.astype(jnp.int32), (0, NP - N),
                      constant_values=G).reshape(NP, 1)
  zrows = jnp.zeros((128, D), jnp.float32)

  deg_part = _deg_call(dst_t)
  h2, starts_out = _h2_call(x_pad, W_conv, deg_part, batch_col)
  p0, p1 = _agg_call(h2, src_t, dst_t, zrows)
  x1 = _x1_call(p0, p1, h2, deg_part, b_conv.reshape(1, D))

  starts_ext = jnp.concatenate(
      [starts_out[0], jnp.full((16,), N, jnp.int32)])
  first_idx = starts_out[1]
  pooled, news_rows = _pool_call(x1, x_pad, starts_ext, first_idx)

  W1a, W1b = W1[:D], W1[D:]
  W2p = jnp.pad(W2, ((0, 0), (0, D - 2)))
  b2p = jnp.pad(b2, (0, D - 2))
  out = _head_call(pooled, news_rows, W0, b0.reshape(1, D),
                   W1a, W1b, b1.reshape(1, D), W2p, b2p.reshape(1, D))
  return out[:, :2]


# speedup vs baseline: 1.3206x; 1.3206x over previous
import functools

import jax
import jax.numpy as jnp
from jax import lax
from jax.experimental import pallas as pl
from jax.experimental.pallas import tpu as pltpu
from jax.experimental.pallas import tpu_sc as plsc

N = 10000
E = 320000
D = 128
G = 128
NP = 10240
NC = 2
NS = 16
NW = NC * NS
EPT = 10112
K = 128
NCH = EPT // K
BR = 512
NBLK = NP // BR

_mesh = functools.partial(
    plsc.VectorSubcoreMesh, core_axis_name="c", subcore_axis_name="s")


def _deg_body(dst_hbm, deg_out, dstbuf, acc):
  c = lax.axis_index("c")
  s = lax.axis_index("s")
  w = s * NC + c

  def zero(i, _):
    acc[pl.ds(i * 16, 16)] = jnp.zeros((16,), jnp.float32)
    return 0
  lax.fori_loop(0, NP // 16, zero, 0)

  pltpu.sync_copy(dst_hbm.at[w], dstbuf)
  ones = jnp.ones((16,), jnp.float32)

  def body(j, _):
    idx = dstbuf[pl.ds(j * 16, 16)]
    plsc.addupdate_scatter(acc, [idx], ones)
    return 0
  lax.fori_loop(0, EPT // 16, body, 0)
  pltpu.sync_copy(acc, deg_out.at[w])


def _deg_call(dst_t):
  kern = pl.kernel(
      _deg_body,
      out_type=jax.ShapeDtypeStruct((NW, NP), jnp.float32),
      mesh=_mesh(),
      scratch_types=[
          pltpu.VMEM((EPT,), jnp.int32),
          pltpu.VMEM((NP,), jnp.float32),
      ],
      compiler_params=pltpu.CompilerParams(needs_layout_passes=False),
  )
  return kern(dst_t)


def _h2_body(x_ref, w_ref, deg_ref, batch_ref, h2_ref, starts_ref):
  ones = jnp.ones((NW, 1), jnp.float32)
  degs = lax.dot_general(deg_ref[...], ones,
                         (((0,), (0,)), ((), ()))) + 1.0
  dinv = lax.rsqrt(degs)
  h = jnp.dot(x_ref[...], w_ref[...], preferred_element_type=jnp.float32)
  h2_ref[...] = h * dinv

  @pl.when(pl.program_id(0) == 0)
  def _():
    b = batch_ref[...]
    gid = lax.broadcasted_iota(jnp.int32, (1, G), 1)
    oh = (b == gid).astype(jnp.float32)
    hist = jnp.sum(oh, axis=0)
    r = lax.broadcasted_iota(jnp.int32, (G, G), 0)
    cc = lax.broadcasted_iota(jnp.int32, (G, G), 1)
    ut = (r < cc).astype(jnp.float32)
    starts = lax.dot_general(hist, ut, (((0,), (0,)), ((), ())))
    starts_i = starts.astype(jnp.int32)
    first = jnp.minimum(starts_i, N - 1)
    pad = jnp.full((6, G), N, jnp.int32)
    starts_ref[...] = jnp.concatenate(
        [starts_i[None], first[None], pad], axis=0)


def _h2_call(x_pad, W_conv, deg_part, batch_col):
  return pl.pallas_call(
      _h2_body,
      grid=(NBLK,),
      in_specs=[
          pl.BlockSpec((BR, D), lambda i: (i, 0)),
          pl.BlockSpec((D, D), lambda i: (0, 0)),
          pl.BlockSpec((NW, BR), lambda i: (0, i)),
          pl.BlockSpec((NP, 1), lambda i: (0, 0)),
      ],
      out_specs=[
          pl.BlockSpec((BR, D), lambda i: (i, 0)),
          pl.BlockSpec((8, G), lambda i: (0, 0)),
      ],
      out_shape=[
          jax.ShapeDtypeStruct((NP, D), jnp.float32),
          jax.ShapeDtypeStruct((8, G), jnp.int32),
      ],
  )(x_pad, W_conv, deg_part, batch_col)


def _agg_body(h2_hbm, src_hbm, dst_hbm, zrows_hbm, p0, p1,
              src_v, dst_v, rows, acc, gsem):
  c = lax.axis_index("c")
  s = lax.axis_index("s")
  w = s * NC + c
  rpt = NP // NS
  r0 = s * rpt
  zk = 128

  for k in range(rpt // zk):
    pltpu.sync_copy(zrows_hbm, acc.at[pl.ds(r0 + k * zk, zk)])
  plsc.subcore_barrier()

  def chunk(j, _):
    base = j * K
    pltpu.sync_copy(src_hbm.at[w, pl.ds(base, K)], src_v)
    pltpu.sync_copy(dst_hbm.at[w, pl.ds(base, K)], dst_v)
    pltpu.async_copy(h2_hbm.at[src_v], rows, gsem).wait()
    pltpu.sync_copy(rows, acc.at[dst_v], add=True)
    return 0
  lax.fori_loop(0, NCH, chunk, 0)
  plsc.subcore_barrier()

  sl = pl.ds(r0, rpt)

  @pl.when(c == 0)
  def _():
    pltpu.sync_copy(acc.at[sl], p0.at[sl])

  @pl.when(c == 1)
  def _():
    pltpu.sync_copy(acc.at[sl], p1.at[sl])


def _agg_call(h2, src_t, dst_t, zrows):
  kern = pl.kernel(
      _agg_body,
      out_type=(
          jax.ShapeDtypeStruct((NP, D), jnp.float32),
          jax.ShapeDtypeStruct((NP, D), jnp.float32),
      ),
      mesh=_mesh(),
      scratch_types=[
          pltpu.VMEM((K,), jnp.int32),
          pltpu.VMEM((K,), jnp.int32),
          pltpu.VMEM((K, D), jnp.float32),
          pltpu.VMEM_SHARED((NP, D), jnp.float32),
          pltpu.SemaphoreType.DMA,
      ],
  )
  return kern(h2, src_t, dst_t, zrows)


def _x1_body(p0_ref, p1_ref, h2_ref, deg_ref, b_ref, x1_ref):
  ones = jnp.ones((NW, 1), jnp.float32)
  degs = lax.dot_general(deg_ref[...], ones,
                         (((0,), (0,)), ((), ()))) + 1.0
  dinv = lax.rsqrt(degs)
  agg = p0_ref[...] + p1_ref[...] + h2_ref[...]
  x1_ref[...] = jnp.maximum(dinv * agg + b_ref[...], 0.0)


def _x1_call(p0, p1, h2, deg_part, b_conv_row):
  blk = pl.BlockSpec((BR, D), lambda i: (i, 0))
  return pl.pallas_call(
      _x1_body,
      grid=(NBLK,),
      in_specs=[
          blk,
          blk,
          blk,
          pl.BlockSpec((NW, BR), lambda i: (0, i)),
          pl.BlockSpec((1, D), lambda i: (0, 0)),
      ],
      out_specs=pl.BlockSpec((BR, D), lambda i: (i, 0)),
      out_shape=jax.ShapeDtypeStruct((NP, D), jnp.float32),
  )(p0, p1, h2, deg_part, b_conv_row)


def _pool_body(x1_hbm, x_hbm, starts_hbm, first_hbm, pooled, news,
               starts_v, first_v, buf, rowbuf, newsbuf):
  c = lax.axis_index("c")
  s = lax.axis_index("s")
  w = s * NC + c
  pltpu.sync_copy(starts_hbm, starts_v)
  pltpu.sync_copy(first_hbm, first_v)
  lane = lax.iota(jnp.int32, 16)

  def extract(ref, idx):
    vec = ref[pl.ds((idx // 16) * 16, 16)]
    return jnp.max(jnp.where(lane == (idx % 16), vec, -1))

  for jj in range(G // NW):
    g = w * (G // NW) + jj
    sval = extract(starts_v, g)
    eval_ = extract(starts_v, g + 1)
    fi = extract(first_v, g)

    pltpu.sync_copy(x_hbm.at[fi], newsbuf)
    pltpu.sync_copy(newsbuf, news.at[g])

    rowa = (sval // K) * K
    init_m = tuple(jnp.full((16,), -jnp.inf, jnp.float32) for _ in range(8))

    def cond(carry):
      return rowa + carry[0] * K < eval_

    def body(carry):
      k, m = carry
      row0 = rowa + k * K
      pltpu.sync_copy(x1_hbm.at[pl.ds(row0, K)], buf)

      def inner(r, m):
        gr = row0 + r
        valid = jnp.logical_and(gr >= sval, gr < eval_)
        return tuple(
            jnp.where(valid,
                      jnp.maximum(m[cc], buf[r, pl.ds(cc * 16, 16)]),
                      m[cc])
            for cc in range(8))
      m = lax.fori_loop(0, K, inner, m)
      return (k + 1, m)

    _, m = lax.while_loop(cond, body, (jnp.int32(0), init_m))
    for cc in range(8):
      rowbuf[pl.ds(cc * 16, 16)] = m[cc]
    pltpu.sync_copy(rowbuf, pooled.at[g])


def _pool_call(x1, x_pad, starts_ext, first_idx):
  kern = pl.kernel(
      _pool_body,
      out_type=(
          jax.ShapeDtypeStruct((G, D), jnp.float32),
          jax.ShapeDtypeStruct((G, D), jnp.float32),
      ),
      mesh=_mesh(),
      scratch_types=[
          pltpu.VMEM((144,), jnp.int32),
          pltpu.VMEM((G,), jnp.int32),
          pltpu.VMEM((K, D), jnp.float32),
          pltpu.VMEM((D,), jnp.float32),
          pltpu.VMEM((D,), jnp.float32),
      ],
      compiler_params=pltpu.CompilerParams(needs_layout_passes=False),
  )
  return kern(x1, x_pad, starts_ext, first_idx)


def _head_body(pooled_ref, newsr_ref, w0_ref, b0_ref, w1a_ref, w1b_ref,
               b1_ref, w2_ref, b2_ref, out_ref):
  nr = jnp.dot(newsr_ref[...], w0_ref[...],
               preferred_element_type=jnp.float32) + b0_ref[...]
  nr = jnp.maximum(nr, 0.0)
  z = (jnp.dot(pooled_ref[...], w1a_ref[...],
               preferred_element_type=jnp.float32)
       + jnp.dot(nr, w1b_ref[...], preferred_element_type=jnp.float32)
       + b1_ref[...])
  z = jnp.maximum(z, 0.0)
  logits = jnp.dot(z, w2_ref[...],
                   preferred_element_type=jnp.float32) + b2_ref[...]
  colmask = lax.broadcasted_iota(jnp.int32, (G, D), 1) < 2
  lm = jnp.where(colmask, logits, -1e30)
  mx = jnp.max(lm, axis=1, keepdims=True)
  sh = lm - mx
  se = jnp.sum(jnp.exp(sh), axis=1, keepdims=True)
  out_ref[...] = sh - jnp.log(se)


def _head_call(pooled, news_rows, W0, b0r, W1a, W1b, b1r, W2p, b2r):
  return pl.pallas_call(
      _head_body,
      out_shape=jax.ShapeDtypeStruct((G, D), jnp.float32),
  )(pooled, news_rows, W0, b0r, W1a, W1b, b1r, W2p, b2r)


@jax.jit
def kernel(x, edge_index, batch, num_graphs,
           W_conv, b_conv, W0, b0, W1, b1, W2, b2):
  del num_graphs
  src = edge_index[0].astype(jnp.int32).reshape(NW, E // NW)
  dst = edge_index[1].astype(jnp.int32).reshape(NW, E // NW)
  padw = EPT - E // NW
  src_t = jnp.concatenate(
      [src, jnp.zeros((NW, padw), jnp.int32)], axis=1)
  dst_pad = jnp.broadcast_to(
      N + jnp.arange(padw, dtype=jnp.int32) % (NP - N), (NW, padw))
  dst_t = jnp.concatenate([dst, dst_pad], axis=1)
  x_pad = jnp.pad(x, ((0, NP - N), (0, 0)))
  batch_col = jnp.pad(batch.astype(jnp.int32), (0, NP - N),
                      constant_values=G).reshape(NP, 1)
  zrows = jnp.zeros((128, D), jnp.float32)

  deg_part = _deg_call(dst_t)
  h2, starts_out = _h2_call(x_pad, W_conv, deg_part, batch_col)
  p0, p1 = _agg_call(h2, src_t, dst_t, zrows)
  x1 = _x1_call(p0, p1, h2, deg_part, b_conv.reshape(1, D))

  starts_ext = jnp.concatenate(
      [starts_out[0], jnp.full((16,), N, jnp.int32)])
  first_idx = starts_out[1]
  pooled, news_rows = _pool_call(x1, x_pad, starts_ext, first_idx)

  W1a, W1b = W1[:D], W1[D:]
  W2p = jnp.pad(W2, ((0, 0), (0, D - 2)))
  b2p = jnp.pad(b2, (0, D - 2))
  out = _head_call(pooled, news_rows, W0, b0.reshape(1, D),
                   W1a, W1b, b1.reshape(1, D), W2p, b2p.reshape(1, D))
  return out[:, :2]


# speedup vs baseline: 1.5255x; 1.1552x over previous
import functools

import jax
import jax.numpy as jnp
from jax import lax
from jax.experimental import pallas as pl
from jax.experimental.pallas import tpu as pltpu
from jax.experimental.pallas import tpu_sc as plsc

N = 10000
E = 320000
D = 128
G = 128
NP = 10240
NC = 2
NS = 16
NW = NC * NS
EPT = 10112
K = 128
NCH = EPT // K
BR = 512
NBLK = NP // BR

_mesh = functools.partial(
    plsc.VectorSubcoreMesh, core_axis_name="c", subcore_axis_name="s")


def _deg_body(dst_hbm, deg_out, dstbuf, acc):
  c = lax.axis_index("c")
  s = lax.axis_index("s")
  w = s * NC + c

  def zero(i, _):
    acc[pl.ds(i * 16, 16)] = jnp.zeros((16,), jnp.float32)
    return 0
  lax.fori_loop(0, NP // 16, zero, 0)

  pltpu.sync_copy(dst_hbm.at[w], dstbuf)
  ones = jnp.ones((16,), jnp.float32)

  def body(j, _):
    idx = dstbuf[pl.ds(j * 16, 16)]
    plsc.addupdate_scatter(acc, [idx], ones)
    return 0
  lax.fori_loop(0, EPT // 16, body, 0)
  pltpu.sync_copy(acc, deg_out.at[w])


def _deg_call(dst_t):
  kern = pl.kernel(
      _deg_body,
      out_type=jax.ShapeDtypeStruct((NW, NP), jnp.float32),
      mesh=_mesh(),
      scratch_types=[
          pltpu.VMEM((EPT,), jnp.int32),
          pltpu.VMEM((NP,), jnp.float32),
      ],
      compiler_params=pltpu.CompilerParams(needs_layout_passes=False),
  )
  return kern(dst_t)


def _h2_body(x_ref, w_ref, deg_ref, batch_ref, h2_ref, starts_ref):
  ones = jnp.ones((NW, 1), jnp.float32)
  degs = lax.dot_general(deg_ref[...], ones,
                         (((0,), (0,)), ((), ()))) + 1.0
  dinv = lax.rsqrt(degs)
  h = jnp.dot(x_ref[...], w_ref[...], preferred_element_type=jnp.float32)
  h2_ref[...] = h * dinv

  @pl.when(pl.program_id(0) == 0)
  def _():
    b = batch_ref[...]
    gid = lax.broadcasted_iota(jnp.int32, (1, G), 1)
    oh = (b == gid).astype(jnp.float32)
    hist = jnp.sum(oh, axis=0)
    r = lax.broadcasted_iota(jnp.int32, (G, G), 0)
    cc = lax.broadcasted_iota(jnp.int32, (G, G), 1)
    ut = (r < cc).astype(jnp.float32)
    starts = lax.dot_general(hist, ut, (((0,), (0,)), ((), ())))
    starts_i = starts.astype(jnp.int32)
    first = jnp.minimum(starts_i, N - 1)
    pad = jnp.full((6, G), N, jnp.int32)
    starts_ref[...] = jnp.concatenate(
        [starts_i[None], first[None], pad], axis=0)


def _h2_call(x_pad, W_conv, deg_part, batch_col):
  return pl.pallas_call(
      _h2_body,
      grid=(NBLK,),
      in_specs=[
          pl.BlockSpec((BR, D), lambda i: (i, 0)),
          pl.BlockSpec((D, D), lambda i: (0, 0)),
          pl.BlockSpec((NW, BR), lambda i: (0, i)),
          pl.BlockSpec((NP, 1), lambda i: (0, 0)),
      ],
      out_specs=[
          pl.BlockSpec((BR, D), lambda i: (i, 0)),
          pl.BlockSpec((8, G), lambda i: (0, 0)),
      ],
      out_shape=[
          jax.ShapeDtypeStruct((NP, D), jnp.float32),
          jax.ShapeDtypeStruct((8, G), jnp.int32),
      ],
  )(x_pad, W_conv, deg_part, batch_col)


def _agg_body(h2_hbm, src_hbm, dst_hbm, zrows_hbm, p0, p1,
              src_v, dst_v, rows, acc, gsem):
  c = lax.axis_index("c")
  s = lax.axis_index("s")
  w = s * NC + c
  rpt = NP // NS
  r0 = s * rpt
  zk = 128

  for k in range(rpt // zk):
    pltpu.sync_copy(zrows_hbm, acc.at[pl.ds(r0 + k * zk, zk)])
  pltpu.sync_copy(src_hbm.at[w], src_v)
  pltpu.sync_copy(dst_hbm.at[w], dst_v)
  plsc.subcore_barrier()

  def chunk(j, _):
    pltpu.async_copy(h2_hbm.at[src_v.at[pl.ds(j * K, K)]], rows, gsem).wait()
    pltpu.sync_copy(rows, acc.at[dst_v.at[j]], add=True)
    return 0
  lax.fori_loop(0, NCH, chunk, 0)
  plsc.subcore_barrier()

  sl = pl.ds(r0, rpt)

  @pl.when(c == 0)
  def _():
    pltpu.sync_copy(acc.at[sl], p0.at[sl])

  @pl.when(c == 1)
  def _():
    pltpu.sync_copy(acc.at[sl], p1.at[sl])


def _agg_call(h2, src_t, dst_t, zrows):
  kern = pl.kernel(
      _agg_body,
      out_type=(
          jax.ShapeDtypeStruct((NP, D), jnp.float32),
          jax.ShapeDtypeStruct((NP, D), jnp.float32),
      ),
      mesh=_mesh(),
      scratch_types=[
          pltpu.VMEM((EPT,), jnp.int32),
          pltpu.VMEM((NCH, K), jnp.int32),
          pltpu.VMEM((K, D), jnp.float32),
          pltpu.VMEM_SHARED((NP, D), jnp.float32),
          pltpu.SemaphoreType.DMA,
      ],
  )
  return kern(h2, src_t, dst_t.reshape(NW, NCH, K), zrows)


def _x1_body(p0_ref, p1_ref, h2_ref, deg_ref, b_ref, x1_ref):
  ones = jnp.ones((NW, 1), jnp.float32)
  degs = lax.dot_general(deg_ref[...], ones,
                         (((0,), (0,)), ((), ()))) + 1.0
  dinv = lax.rsqrt(degs)
  agg = p0_ref[...] + p1_ref[...] + h2_ref[...]
  x1_ref[...] = jnp.maximum(dinv * agg + b_ref[...], 0.0)


def _x1_call(p0, p1, h2, deg_part, b_conv_row):
  blk = pl.BlockSpec((BR, D), lambda i: (i, 0))
  return pl.pallas_call(
      _x1_body,
      grid=(NBLK,),
      in_specs=[
          blk,
          blk,
          blk,
          pl.BlockSpec((NW, BR), lambda i: (0, i)),
          pl.BlockSpec((1, D), lambda i: (0, 0)),
      ],
      out_specs=pl.BlockSpec((BR, D), lambda i: (i, 0)),
      out_shape=jax.ShapeDtypeStruct((NP, D), jnp.float32),
  )(p0, p1, h2, deg_part, b_conv_row)


def _pool_body(x1_hbm, x_hbm, starts_hbm, first_hbm, pooled, news,
               starts_v, first_v, buf, rowbuf, newsbuf):
  c = lax.axis_index("c")
  s = lax.axis_index("s")
  w = s * NC + c
  pltpu.sync_copy(starts_hbm, starts_v)
  pltpu.sync_copy(first_hbm, first_v)
  lane = lax.iota(jnp.int32, 16)

  def extract(ref, idx):
    vec = ref[pl.ds((idx // 16) * 16, 16)]
    return jnp.max(jnp.where(lane == (idx % 16), vec, -1))

  for jj in range(G // NW):
    g = w * (G // NW) + jj
    sval = extract(starts_v, g)
    eval_ = extract(starts_v, g + 1)
    fi = extract(first_v, g)

    pltpu.sync_copy(x_hbm.at[fi], newsbuf)
    pltpu.sync_copy(newsbuf, news.at[g])

    rowa = (sval // K) * K
    init_m = tuple(jnp.full((16,), -jnp.inf, jnp.float32) for _ in range(8))

    def cond(carry):
      return rowa + carry[0] * K < eval_

    def body(carry):
      k, m = carry
      row0 = rowa + k * K
      pltpu.sync_copy(x1_hbm.at[pl.ds(row0, K)], buf)

      def inner(r, m):
        gr = row0 + r
        valid = jnp.logical_and(gr >= sval, gr < eval_)
        return tuple(
            jnp.where(valid,
                      jnp.maximum(m[cc], buf[r, pl.ds(cc * 16, 16)]),
                      m[cc])
            for cc in range(8))
      m = lax.fori_loop(0, K, inner, m)
      return (k + 1, m)

    _, m = lax.while_loop(cond, body, (jnp.int32(0), init_m))
    for cc in range(8):
      rowbuf[pl.ds(cc * 16, 16)] = m[cc]
    pltpu.sync_copy(rowbuf, pooled.at[g])


def _pool_call(x1, x_pad, starts_ext, first_idx):
  kern = pl.kernel(
      _pool_body,
      out_type=(
          jax.ShapeDtypeStruct((G, D), jnp.float32),
          jax.ShapeDtypeStruct((G, D), jnp.float32),
      ),
      mesh=_mesh(),
      scratch_types=[
          pltpu.VMEM((144,), jnp.int32),
          pltpu.VMEM((G,), jnp.int32),
          pltpu.VMEM((K, D), jnp.float32),
          pltpu.VMEM((D,), jnp.float32),
          pltpu.VMEM((D,), jnp.float32),
      ],
      compiler_params=pltpu.CompilerParams(needs_layout_passes=False),
  )
  return kern(x1, x_pad, starts_ext, first_idx)


def _head_body(pooled_ref, newsr_ref, w0_ref, b0_ref, w1a_ref, w1b_ref,
               b1_ref, w2_ref, b2_ref, out_ref):
  nr = jnp.dot(newsr_ref[...], w0_ref[...],
               preferred_element_type=jnp.float32) + b0_ref[...]
  nr = jnp.maximum(nr, 0.0)
  z = (jnp.dot(pooled_ref[...], w1a_ref[...],
               preferred_element_type=jnp.float32)
       + jnp.dot(nr, w1b_ref[...], preferred_element_type=jnp.float32)
       + b1_ref[...])
  z = jnp.maximum(z, 0.0)
  logits = jnp.dot(z, w2_ref[...],
                   preferred_element_type=jnp.float32) + b2_ref[...]
  colmask = lax.broadcasted_iota(jnp.int32, (G, D), 1) < 2
  lm = jnp.where(colmask, logits, -1e30)
  mx = jnp.max(lm, axis=1, keepdims=True)
  sh = lm - mx
  se = jnp.sum(jnp.exp(sh), axis=1, keepdims=True)
  out_ref[...] = sh - jnp.log(se)


def _head_call(pooled, news_rows, W0, b0r, W1a, W1b, b1r, W2p, b2r):
  return pl.pallas_call(
      _head_body,
      out_shape=jax.ShapeDtypeStruct((G, D), jnp.float32),
  )(pooled, news_rows, W0, b0r, W1a, W1b, b1r, W2p, b2r)


@jax.jit
def kernel(x, edge_index, batch, num_graphs,
           W_conv, b_conv, W0, b0, W1, b1, W2, b2):
  del num_graphs
  src = edge_index[0].astype(jnp.int32).reshape(NW, E // NW)
  dst = edge_index[1].astype(jnp.int32).reshape(NW, E // NW)
  padw = EPT - E // NW
  src_t = jnp.concatenate(
      [src, jnp.zeros((NW, padw), jnp.int32)], axis=1)
  dst_pad = jnp.broadcast_to(
      N + jnp.arange(padw, dtype=jnp.int32) % (NP - N), (NW, padw))
  dst_t = jnp.concatenate([dst, dst_pad], axis=1)
  x_pad = jnp.pad(x, ((0, NP - N), (0, 0)))
  batch_col = jnp.pad(batch.astype(jnp.int32), (0, NP - N),
                      constant_values=G).reshape(NP, 1)
  zrows = jnp.zeros((128, D), jnp.float32)

  deg_part = _deg_call(dst_t)
  h2, starts_out = _h2_call(x_pad, W_conv, deg_part, batch_col)
  p0, p1 = _agg_call(h2, src_t, dst_t, zrows)
  x1 = _x1_call(p0, p1, h2, deg_part, b_conv.reshape(1, D))

  starts_ext = jnp.concatenate(
      [starts_out[0], jnp.full((16,), N, jnp.int32)])
  first_idx = starts_out[1]
  pooled, news_rows = _pool_call(x1, x_pad, starts_ext, first_idx)

  W1a, W1b = W1[:D], W1[D:]
  W2p = jnp.pad(W2, ((0, 0), (0, D - 2)))
  b2p = jnp.pad(b2, (0, D - 2))
  out = _head_call(pooled, news_rows, W0, b0.reshape(1, D),
                   W1a, W1b, b1.reshape(1, D), W2p, b2p.reshape(1, D))
  return out[:, :2]


# speedup vs baseline: 1.5510x; 1.0168x over previous
import functools

import jax
import jax.numpy as jnp
from jax import lax
from jax.experimental import pallas as pl
from jax.experimental.pallas import tpu as pltpu
from jax.experimental.pallas import tpu_sc as plsc

N = 10000
E = 320000
D = 128
G = 128
NP = 10240
NC = 2
NS = 16
NW = NC * NS
EPT = 10112
K = 128
NCH = EPT // K
BR = 512
NBLK = NP // BR

_mesh = functools.partial(
    plsc.VectorSubcoreMesh, core_axis_name="c", subcore_axis_name="s")


def _deg_body(dst_hbm, deg_out, dstbuf, acc):
  c = lax.axis_index("c")
  s = lax.axis_index("s")
  w = s * NC + c

  def zero(i, _):
    acc[pl.ds(i * 16, 16)] = jnp.zeros((16,), jnp.float32)
    return 0
  lax.fori_loop(0, NP // 16, zero, 0)

  pltpu.sync_copy(dst_hbm.at[w], dstbuf)
  ones = jnp.ones((16,), jnp.float32)

  def body(j, _):
    idx = dstbuf[pl.ds(j * 16, 16)]
    plsc.addupdate_scatter(acc, [idx], ones)
    return 0
  lax.fori_loop(0, EPT // 16, body, 0)
  pltpu.sync_copy(acc, deg_out.at[w])


def _deg_call(dst_t):
  kern = pl.kernel(
      _deg_body,
      out_type=jax.ShapeDtypeStruct((NW, NP), jnp.float32),
      mesh=_mesh(),
      scratch_types=[
          pltpu.VMEM((EPT,), jnp.int32),
          pltpu.VMEM((NP,), jnp.float32),
      ],
      compiler_params=pltpu.CompilerParams(needs_layout_passes=False),
  )
  return kern(dst_t)


def _h2_body(x_ref, w_ref, deg_ref, batch_ref, h2_ref, starts_ref):
  ones = jnp.ones((NW, 1), jnp.float32)
  degs = lax.dot_general(deg_ref[...], ones,
                         (((0,), (0,)), ((), ()))) + 1.0
  dinv = lax.rsqrt(degs)
  h = jnp.dot(x_ref[...], w_ref[...], preferred_element_type=jnp.float32)
  h2_ref[...] = h * dinv

  @pl.when(pl.program_id(0) == 0)
  def _():
    b = batch_ref[...]
    gid = lax.broadcasted_iota(jnp.int32, (1, G), 1)
    oh = (b == gid).astype(jnp.float32)
    hist = jnp.sum(oh, axis=0)
    r = lax.broadcasted_iota(jnp.int32, (G, G), 0)
    cc = lax.broadcasted_iota(jnp.int32, (G, G), 1)
    ut = (r < cc).astype(jnp.float32)
    starts = lax.dot_general(hist, ut, (((0,), (0,)), ((), ())))
    starts_i = starts.astype(jnp.int32)
    first = jnp.minimum(starts_i, N - 1)
    pad = jnp.full((6, G), N, jnp.int32)
    starts_ref[...] = jnp.concatenate(
        [starts_i[None], first[None], pad], axis=0)


def _h2_call(x_pad, W_conv, deg_part, batch_col):
  return pl.pallas_call(
      _h2_body,
      grid=(NBLK,),
      in_specs=[
          pl.BlockSpec((BR, D), lambda i: (i, 0)),
          pl.BlockSpec((D, D), lambda i: (0, 0)),
          pl.BlockSpec((NW, BR), lambda i: (0, i)),
          pl.BlockSpec((NP, 1), lambda i: (0, 0)),
      ],
      out_specs=[
          pl.BlockSpec((BR, D), lambda i: (i, 0)),
          pl.BlockSpec((8, G), lambda i: (0, 0)),
      ],
      out_shape=[
          jax.ShapeDtypeStruct((NP, D), jnp.float32),
          jax.ShapeDtypeStruct((8, G), jnp.int32),
      ],
  )(x_pad, W_conv, deg_part, batch_col)


def _agg_body(h2_hbm, src_hbm, dst_hbm, zrows_hbm, p0, p1,
              src_v, dst_v, rows, rowsb, acc, gsem, gsemb):
  c = lax.axis_index("c")
  s = lax.axis_index("s")
  w = s * NC + c
  rpt = NP // NS
  r0 = s * rpt
  zk = 128

  for k in range(rpt // zk):
    pltpu.sync_copy(zrows_hbm, acc.at[pl.ds(r0 + k * zk, zk)])
  pltpu.sync_copy(dst_hbm.at[w], dst_v)
  plsc.subcore_barrier()

  def round_(r, _):
    j = r * 2
    pltpu.sync_copy(src_hbm.at[w, pl.ds(j * K, 2 * K)], src_v)
    ca = pltpu.make_async_copy(h2_hbm.at[src_v.at[pl.ds(0, K)]], rows, gsem)
    cb = pltpu.make_async_copy(h2_hbm.at[src_v.at[pl.ds(K, K)]], rowsb, gsemb)
    ca.start()
    cb.start()
    ca.wait()
    pltpu.sync_copy(rows, acc.at[dst_v.at[j]], add=True)
    cb.wait()
    pltpu.sync_copy(rowsb, acc.at[dst_v.at[j + 1]], add=True)
    return 0
  lax.fori_loop(0, NCH // 2, round_, 0)

  j_tail = NCH - 1
  pltpu.sync_copy(src_hbm.at[w, pl.ds(j_tail * K, K)], src_v.at[pl.ds(0, K)])
  pltpu.async_copy(h2_hbm.at[src_v.at[pl.ds(0, K)]], rows, gsem).wait()
  pltpu.sync_copy(rows, acc.at[dst_v.at[j_tail]], add=True)
  plsc.subcore_barrier()

  sl = pl.ds(r0, rpt)

  @pl.when(c == 0)
  def _():
    pltpu.sync_copy(acc.at[sl], p0.at[sl])

  @pl.when(c == 1)
  def _():
    pltpu.sync_copy(acc.at[sl], p1.at[sl])


def _agg_call(h2, src_t, dst_t, zrows):
  kern = pl.kernel(
      _agg_body,
      out_type=(
          jax.ShapeDtypeStruct((NP, D), jnp.float32),
          jax.ShapeDtypeStruct((NP, D), jnp.float32),
      ),
      mesh=_mesh(),
      scratch_types=[
          pltpu.VMEM((2 * K,), jnp.int32),
          pltpu.VMEM((NCH, K), jnp.int32),
          pltpu.VMEM((K, D), jnp.float32),
          pltpu.VMEM((K, D), jnp.float32),
          pltpu.VMEM_SHARED((NP, D), jnp.float32),
          pltpu.SemaphoreType.DMA,
          pltpu.SemaphoreType.DMA,
      ],
  )
  return kern(h2, src_t, dst_t.reshape(NW, NCH, K), zrows)


def _x1_body(p0_ref, p1_ref, h2_ref, deg_ref, b_ref, x1_ref):
  ones = jnp.ones((NW, 1), jnp.float32)
  degs = lax.dot_general(deg_ref[...], ones,
                         (((0,), (0,)), ((), ()))) + 1.0
  dinv = lax.rsqrt(degs)
  agg = p0_ref[...] + p1_ref[...] + h2_ref[...]
  x1_ref[...] = jnp.maximum(dinv * agg + b_ref[...], 0.0)


def _x1_call(p0, p1, h2, deg_part, b_conv_row):
  blk = pl.BlockSpec((BR, D), lambda i: (i, 0))
  return pl.pallas_call(
      _x1_body,
      grid=(NBLK,),
      in_specs=[
          blk,
          blk,
          blk,
          pl.BlockSpec((NW, BR), lambda i: (0, i)),
          pl.BlockSpec((1, D), lambda i: (0, 0)),
      ],
      out_specs=pl.BlockSpec((BR, D), lambda i: (i, 0)),
      out_shape=jax.ShapeDtypeStruct((NP, D), jnp.float32),
  )(p0, p1, h2, deg_part, b_conv_row)


def _pool_body(x1_hbm, x_hbm, starts_hbm, first_hbm, pooled, news,
               starts_v, first_v, buf, rowbuf, newsbuf):
  c = lax.axis_index("c")
  s = lax.axis_index("s")
  w = s * NC + c
  pltpu.sync_copy(starts_hbm, starts_v)
  pltpu.sync_copy(first_hbm, first_v)
  lane = lax.iota(jnp.int32, 16)

  def extract(ref, idx):
    vec = ref[pl.ds((idx // 16) * 16, 16)]
    return jnp.max(jnp.where(lane == (idx % 16), vec, -1))

  for jj in range(G // NW):
    g = w * (G // NW) + jj
    sval = extract(starts_v, g)
    eval_ = extract(starts_v, g + 1)
    fi = extract(first_v, g)

    pltpu.sync_copy(x_hbm.at[fi], newsbuf)
    pltpu.sync_copy(newsbuf, news.at[g])

    rowa = (sval // K) * K
    init_m = tuple(jnp.full((16,), -jnp.inf, jnp.float32) for _ in range(8))

    def cond(carry):
      return rowa + carry[0] * K < eval_

    def body(carry):
      k, m = carry
      row0 = rowa + k * K
      pltpu.sync_copy(x1_hbm.at[pl.ds(row0, K)], buf)

      def inner(r, m):
        gr = row0 + r
        valid = jnp.logical_and(gr >= sval, gr < eval_)
        return tuple(
            jnp.where(valid,
                      jnp.maximum(m[cc], buf[r, pl.ds(cc * 16, 16)]),
                      m[cc])
            for cc in range(8))
      m = lax.fori_loop(0, K, inner, m)
      return (k + 1, m)

    _, m = lax.while_loop(cond, body, (jnp.int32(0), init_m))
    for cc in range(8):
      rowbuf[pl.ds(cc * 16, 16)] = m[cc]
    pltpu.sync_copy(rowbuf, pooled.at[g])


def _pool_call(x1, x_pad, starts_ext, first_idx):
  kern = pl.kernel(
      _pool_body,
      out_type=(
          jax.ShapeDtypeStruct((G, D), jnp.float32),
          jax.ShapeDtypeStruct((G, D), jnp.float32),
      ),
      mesh=_mesh(),
      scratch_types=[
          pltpu.VMEM((144,), jnp.int32),
          pltpu.VMEM((G,), jnp.int32),
          pltpu.VMEM((K, D), jnp.float32),
          pltpu.VMEM((D,), jnp.float32),
          pltpu.VMEM((D,), jnp.float32),
      ],
      compiler_params=pltpu.CompilerParams(needs_layout_passes=False),
  )
  return kern(x1, x_pad, starts_ext, first_idx)


def _head_body(pooled_ref, newsr_ref, w0_ref, b0_ref, w1a_ref, w1b_ref,
               b1_ref, w2_ref, b2_ref, out_ref):
  nr = jnp.dot(newsr_ref[...], w0_ref[...],
               preferred_element_type=jnp.float32) + b0_ref[...]
  nr = jnp.maximum(nr, 0.0)
  z = (jnp.dot(pooled_ref[...], w1a_ref[...],
               preferred_element_type=jnp.float32)
       + jnp.dot(nr, w1b_ref[...], preferred_element_type=jnp.float32)
       + b1_ref[...])
  z = jnp.maximum(z, 0.0)
  logits = jnp.dot(z, w2_ref[...],
                   preferred_element_type=jnp.float32) + b2_ref[...]
  colmask = lax.broadcasted_iota(jnp.int32, (G, D), 1) < 2
  lm = jnp.where(colmask, logits, -1e30)
  mx = jnp.max(lm, axis=1, keepdims=True)
  sh = lm - mx
  se = jnp.sum(jnp.exp(sh), axis=1, keepdims=True)
  out_ref[...] = sh - jnp.log(se)


def _head_call(pooled, news_rows, W0, b0r, W1a, W1b, b1r, W2p, b2r):
  return pl.pallas_call(
      _head_body,
      out_shape=jax.ShapeDtypeStruct((G, D), jnp.float32),
  )(pooled, news_rows, W0, b0r, W1a, W1b, b1r, W2p, b2r)


@jax.jit
def kernel(x, edge_index, batch, num_graphs,
           W_conv, b_conv, W0, b0, W1, b1, W2, b2):
  del num_graphs
  src = edge_index[0].astype(jnp.int32).reshape(NW, E // NW)
  dst = edge_index[1].astype(jnp.int32).reshape(NW, E // NW)
  padw = EPT - E // NW
  src_t = jnp.concatenate(
      [src, jnp.zeros((NW, padw), jnp.int32)], axis=1)
  dst_pad = jnp.broadcast_to(
      N + jnp.arange(padw, dtype=jnp.int32) % (NP - N), (NW, padw))
  dst_t = jnp.concatenate([dst, dst_pad], axis=1)
  x_pad = jnp.pad(x, ((0, NP - N), (0, 0)))
  batch_col = jnp.pad(batch.astype(jnp.int32), (0, NP - N),
                      constant_values=G).reshape(NP, 1)
  zrows = jnp.zeros((128, D), jnp.float32)

  deg_part = _deg_call(dst_t)
  h2, starts_out = _h2_call(x_pad, W_conv, deg_part, batch_col)
  p0, p1 = _agg_call(h2, src_t, dst_t, zrows)
  x1 = _x1_call(p0, p1, h2, deg_part, b_conv.reshape(1, D))

  starts_ext = jnp.concatenate(
      [starts_out[0], jnp.full((16,), N, jnp.int32)])
  first_idx = starts_out[1]
  pooled, news_rows = _pool_call(x1, x_pad, starts_ext, first_idx)

  W1a, W1b = W1[:D], W1[D:]
  W2p = jnp.pad(W2, ((0, 0), (0, D - 2)))
  b2p = jnp.pad(b2, (0, D - 2))
  out = _head_call(pooled, news_rows, W0, b0.reshape(1, D),
                   W1a, W1b, b1.reshape(1, D), W2p, b2p.reshape(1, D))
  return out[:, :2]
